# Initial kernel scaffold; baseline (speedup 1.0000x reference)
import jax
import jax.numpy as jnp
from jax.experimental import pallas as pl


def kernel(query, mem_state, Wq, bq, Wk, bk, Wv, bv, Wo, bo):
    raise NotImplementedError("write your pallas kernel here")


# speedup vs baseline: 6.0653x; 6.0653x over previous
import functools
import math

import jax
import jax.numpy as jnp
from jax.experimental import pallas as pl
from jax.experimental.pallas import tpu as pltpu

_TOP_K = 32


def _kv_proj_body(mem_ref, wk_ref, bk_ref, wv_ref, bv_ref, k_ref, v_ref):
    m = mem_ref[0]
    k_ref[0] = jax.lax.dot_general(
        m, wk_ref[...], (((1,), (1,)), ((), ())),
        preferred_element_type=jnp.float32) + bk_ref[...]
    v_ref[0] = jax.lax.dot_general(
        m, wv_ref[...], (((1,), (1,)), ((), ())),
        preferred_element_type=jnp.float32) + bv_ref[...]


def _scores_body(q_in_ref, wq_ref, bq_ref, k_ref, s_ref, *, scale):
    q = jax.lax.dot_general(
        q_in_ref[0], wq_ref[...], (((1,), (1,)), ((), ())),
        preferred_element_type=jnp.float32) + bq_ref[...]
    s_ref[0] = jax.lax.dot_general(
        q, k_ref[0], (((1,), (1,)), ((), ())),
        preferred_element_type=jnp.float32) * scale


def _attend_body(s_ref, v_ref, wo_ref, bo_ref, out_ref):
    s = s_ref[0]
    m = jnp.max(s, axis=-1, keepdims=True)

    def step(_, carry):
        cur, _ = carry
        cm = jnp.max(cur, axis=-1, keepdims=True)
        cur = jnp.where(cur == cm, -jnp.inf, cur)
        return cur, cm

    _, thresh = jax.lax.fori_loop(0, _TOP_K, step, (s, m))
    p = jnp.where(s >= thresh, jnp.exp(s - m), 0.0)
    z = jnp.sum(p, axis=-1, keepdims=True)
    ctx = jax.lax.dot_general(
        p, v_ref[0], (((1,), (0,)), ((), ())),
        preferred_element_type=jnp.float32) / z
    out_ref[0] = jax.lax.dot_general(
        ctx, wo_ref[...], (((1,), (1,)), ((), ())),
        preferred_element_type=jnp.float32) + bo_ref[...]


def kernel(query, mem_state, Wq, bq, Wk, bk, Wv, bv, Wo, bo):
    B, T, D = query.shape
    S = mem_state.shape[1]
    scale = 1.0 / math.sqrt(D)

    bq2, bk2, bv2, bo2 = (b.reshape(1, D) for b in (bq, bk, bv, bo))

    BS = min(1024, S)
    k_mat, v_mat = pl.pallas_call(
        _kv_proj_body,
        grid=(B, S // BS),
        in_specs=[
            pl.BlockSpec((1, BS, D), lambda b, s: (b, s, 0)),
            pl.BlockSpec((D, D), lambda b, s: (0, 0)),
            pl.BlockSpec((1, D), lambda b, s: (0, 0)),
            pl.BlockSpec((D, D), lambda b, s: (0, 0)),
            pl.BlockSpec((1, D), lambda b, s: (0, 0)),
        ],
        out_specs=[
            pl.BlockSpec((1, BS, D), lambda b, s: (b, s, 0)),
            pl.BlockSpec((1, BS, D), lambda b, s: (b, s, 0)),
        ],
        out_shape=[
            jax.ShapeDtypeStruct((B, S, D), jnp.float32),
            jax.ShapeDtypeStruct((B, S, D), jnp.float32),
        ],
    )(mem_state, Wk, bk2, Wv, bv2)

    BT = min(256, T)
    scores = pl.pallas_call(
        functools.partial(_scores_body, scale=scale),
        grid=(B, T // BT),
        in_specs=[
            pl.BlockSpec((1, BT, D), lambda b, t: (b, t, 0)),
            pl.BlockSpec((D, D), lambda b, t: (0, 0)),
            pl.BlockSpec((1, D), lambda b, t: (0, 0)),
            pl.BlockSpec((1, S, D), lambda b, t: (b, 0, 0)),
        ],
        out_specs=pl.BlockSpec((1, BT, S), lambda b, t: (b, t, 0)),
        out_shape=jax.ShapeDtypeStruct((B, T, S), jnp.float32),
    )(query, Wq, bq2, k_mat)

    BA = min(128, T)
    out = pl.pallas_call(
        _attend_body,
        grid=(B, T // BA),
        in_specs=[
            pl.BlockSpec((1, BA, S), lambda b, t: (b, t, 0)),
            pl.BlockSpec((1, S, D), lambda b, t: (b, 0, 0)),
            pl.BlockSpec((D, D), lambda b, t: (0, 0)),
            pl.BlockSpec((1, D), lambda b, t: (0, 0)),
        ],
        out_specs=pl.BlockSpec((1, BA, D), lambda b, t: (b, t, 0)),
        out_shape=jax.ShapeDtypeStruct((B, T, D), jnp.float32),
    )(scores, v_mat, Wo, bo2)
    return out


# speedup vs baseline: 6.0757x; 1.0017x over previous
import functools
import math

import jax
import jax.numpy as jnp
from jax.experimental import pallas as pl
from jax.experimental.pallas import tpu as pltpu

_TOP_K = 32


def _kv_proj_body(mem_ref, wk_ref, bk_ref, wv_ref, bv_ref, k_ref, v_ref):
    m = mem_ref[0]
    k_ref[0] = jax.lax.dot_general(
        m, wk_ref[...], (((1,), (1,)), ((), ())),
        preferred_element_type=jnp.float32) + bk_ref[...]
    v_ref[0] = (jax.lax.dot_general(
        m, wv_ref[...], (((1,), (1,)), ((), ())),
        preferred_element_type=jnp.float32) + bv_ref[...]).astype(jnp.bfloat16)


def _scores_body(q_in_ref, wq_ref, bq_ref, k_ref, s_ref, *, scale):
    q = jax.lax.dot_general(
        q_in_ref[0], wq_ref[...], (((1,), (1,)), ((), ())),
        preferred_element_type=jnp.float32) + bq_ref[...]
    s_ref[0] = jax.lax.dot_general(
        q, k_ref[0], (((1,), (1,)), ((), ())),
        preferred_element_type=jnp.float32) * scale


def _attend_body(s_ref, v_ref, wo_ref, bo_ref, out_ref):
    s = s_ref[0]
    m = jnp.max(s, axis=-1, keepdims=True)

    def step(_, carry):
        cur, _ = carry
        cm = jnp.max(cur, axis=-1, keepdims=True)
        cur = jnp.where(cur == cm, -jnp.inf, cur)
        return cur, cm

    _, thresh = jax.lax.fori_loop(0, _TOP_K, step, (s, m))
    p = jnp.where(s >= thresh, jnp.exp(s - m), 0.0)
    z = jnp.sum(p, axis=-1, keepdims=True)
    ctx = jax.lax.dot_general(
        p.astype(jnp.bfloat16), v_ref[0], (((1,), (0,)), ((), ())),
        preferred_element_type=jnp.float32) / z
    out_ref[0] = jax.lax.dot_general(
        ctx.astype(jnp.bfloat16), wo_ref[...], (((1,), (1,)), ((), ())),
        preferred_element_type=jnp.float32) + bo_ref[...]


def kernel(query, mem_state, Wq, bq, Wk, bk, Wv, bv, Wo, bo):
    B, T, D = query.shape
    S = mem_state.shape[1]
    scale = 1.0 / math.sqrt(D)

    bq2, bk2, bv2, bo2 = (b.reshape(1, D) for b in (bq, bk, bv, bo))

    BS = min(1024, S)
    k_mat, v_mat = pl.pallas_call(
        _kv_proj_body,
        grid=(B, S // BS),
        in_specs=[
            pl.BlockSpec((1, BS, D), lambda b, s: (b, s, 0)),
            pl.BlockSpec((D, D), lambda b, s: (0, 0)),
            pl.BlockSpec((1, D), lambda b, s: (0, 0)),
            pl.BlockSpec((D, D), lambda b, s: (0, 0)),
            pl.BlockSpec((1, D), lambda b, s: (0, 0)),
        ],
        out_specs=[
            pl.BlockSpec((1, BS, D), lambda b, s: (b, s, 0)),
            pl.BlockSpec((1, BS, D), lambda b, s: (b, s, 0)),
        ],
        out_shape=[
            jax.ShapeDtypeStruct((B, S, D), jnp.float32),
            jax.ShapeDtypeStruct((B, S, D), jnp.bfloat16),
        ],
    )(mem_state, Wk, bk2, Wv, bv2)

    BT = min(256, T)
    scores = pl.pallas_call(
        functools.partial(_scores_body, scale=scale),
        grid=(B, T // BT),
        in_specs=[
            pl.BlockSpec((1, BT, D), lambda b, t: (b, t, 0)),
            pl.BlockSpec((D, D), lambda b, t: (0, 0)),
            pl.BlockSpec((1, D), lambda b, t: (0, 0)),
            pl.BlockSpec((1, S, D), lambda b, t: (b, 0, 0)),
        ],
        out_specs=pl.BlockSpec((1, BT, S), lambda b, t: (b, t, 0)),
        out_shape=jax.ShapeDtypeStruct((B, T, S), jnp.float32),
    )(query, Wq, bq2, k_mat)

    BA = min(128, T)
    out = pl.pallas_call(
        _attend_body,
        grid=(B, T // BA),
        in_specs=[
            pl.BlockSpec((1, BA, S), lambda b, t: (b, t, 0)),
            pl.BlockSpec((1, S, D), lambda b, t: (b, 0, 0)),
            pl.BlockSpec((D, D), lambda b, t: (0, 0)),
            pl.BlockSpec((1, D), lambda b, t: (0, 0)),
        ],
        out_specs=pl.BlockSpec((1, BA, D), lambda b, t: (b, t, 0)),
        out_shape=jax.ShapeDtypeStruct((B, T, D), jnp.float32),
    )(scores, v_mat, Wo.astype(jnp.bfloat16), bo2)
    return out
